# baseline (speedup 1.0000x reference)
import jax
import jax.numpy as jnp
from jax import lax
from jax.experimental import pallas as pl
from jax.experimental.pallas import tpu as pltpu
from jax.experimental.pallas import tpu_sc as plsc

B = 16384
F = 26
D = 16
FIELD = 38462
V = F * FIELD
VP = V + 4

NC = 2
NS = 16
NW = NC * NS
LANES = 16

C = 512
NFULL = 1952
CPT = NFULL // NW
TAIL0 = NFULL * C
TAILW = 584

BT = B // 128
UNITS = F * BT
UPW = UNITS // NW
G = 4
GPW = UPW // G


def _tr_chunk(slab, lin, dvec):
    for v in range(C):
        vvec = jnp.full((LANES,), v, jnp.int32)
        lin[v, :] = plsc.load_gather(slab, [dvec, vvec])


def _tr_body(tabT, tabflat, out_hbm, slab0, slab1, lin0, lin1, tslab, tlin,
             lin4, sl0, sl1, sw0, sw1, st):
    wid = lax.axis_index("s") * NC + lax.axis_index("c")
    dvec = lax.iota(jnp.int32, LANES)

    def v0_of(c):
        return (c * NW + wid) * C

    pltpu.make_async_copy(tabT.at[:, pl.ds(v0_of(0), C)], slab0, sl0).start()

    def pair(k, carry):
        a = 2 * k
        pltpu.make_async_copy(tabT.at[:, pl.ds(v0_of(a + 1), C)], slab1, sl1).start()

        pltpu.make_async_copy(tabT.at[:, pl.ds(v0_of(a), C)], slab0, sl0).wait()

        @pl.when(k > 0)
        def _():
            pltpu.make_async_copy(lin0, out_hbm.at[pl.ds(0, C), :], sw0).wait()

        _tr_chunk(slab0, lin0, dvec)
        pltpu.make_async_copy(lin0, out_hbm.at[pl.ds(v0_of(a), C), :], sw0).start()

        pltpu.make_async_copy(tabT.at[:, pl.ds(v0_of(a + 2), C)], slab0, sl0).start()

        pltpu.make_async_copy(tabT.at[:, pl.ds(v0_of(a + 1), C)], slab1, sl1).wait()

        @pl.when(k > 0)
        def _():
            pltpu.make_async_copy(lin1, out_hbm.at[pl.ds(0, C), :], sw1).wait()

        _tr_chunk(slab1, lin1, dvec)
        pltpu.make_async_copy(lin1, out_hbm.at[pl.ds(v0_of(a + 1), C), :], sw1).start()
        return carry

    lax.fori_loop(0, CPT // 2, pair, 0)

    last = CPT - 1
    pltpu.make_async_copy(tabT.at[:, pl.ds(v0_of(last), C)], slab0, sl0).wait()
    pltpu.make_async_copy(lin0, out_hbm.at[pl.ds(0, C), :], sw0).wait()
    _tr_chunk(slab0, lin0, dvec)
    pltpu.sync_copy(lin0, out_hbm.at[pl.ds(v0_of(last), C), :])
    pltpu.make_async_copy(lin1, out_hbm.at[pl.ds(0, C), :], sw1).wait()

    @pl.when(wid == NW - 1)
    def _():
        pltpu.sync_copy(tabT.at[:, pl.ds(TAIL0, TAILW)], tslab)
        for v in range(TAILW):
            vvec = jnp.full((LANES,), v, jnp.int32)
            tlin[v, :] = plsc.load_gather(tslab, [dvec, vvec])
        pltpu.sync_copy(tlin, out_hbm.at[pl.ds(TAIL0, TAILW), :])
        dflat = dvec * V
        for i in range(4):
            pltpu.async_copy(tabflat.at[dflat + (TAIL0 + TAILW + i)],
                             lin4.at[i], st).wait()
        pltpu.sync_copy(lin4, out_hbm.at[pl.ds(TAIL0 + TAILW, 4), :])


@jax.jit
def _transpose_table(tabT, tabflat):
    mesh = plsc.VectorSubcoreMesh(
        core_axis_name="c", subcore_axis_name="s", num_cores=NC, num_subcores=NS
    )
    return pl.kernel(
        _tr_body,
        out_type=jax.ShapeDtypeStruct((VP, D), jnp.float32),
        mesh=mesh,
        compiler_params=pltpu.CompilerParams(
            use_tc_tiling_on_sc=False, needs_layout_passes=False
        ),
        scratch_types=[
            pltpu.VMEM((D, C), jnp.float32),
            pltpu.VMEM((D, C), jnp.float32),
            pltpu.VMEM((C, D), jnp.float32),
            pltpu.VMEM((C, D), jnp.float32),
            pltpu.VMEM((D, TAILW), jnp.float32),
            pltpu.VMEM((TAILW, D), jnp.float32),
            pltpu.VMEM((4, D), jnp.float32),
            pltpu.SemaphoreType.DMA,
            pltpu.SemaphoreType.DMA,
            pltpu.SemaphoreType.DMA,
            pltpu.SemaphoreType.DMA,
            pltpu.SemaphoreType.DMA,
        ],
    )(tabT, tabflat)


def _unit_f_bt(base, g, j):
    u = base + g * G + j
    return u // BT, u % BT


def _prep_idx(idxall, idxg, base, g):
    for j in range(G):
        f, _ = _unit_f_bt(base, g, j)
        off = f * FIELD
        for j2 in range(128 // LANES):
            s = pl.ds(g * (G * 128) + j * 128 + j2 * LANES, LANES)
            d = pl.ds(j * 128 + j2 * LANES, LANES)
            idxg[d] = idxall[s] + off


def _transpose_unit(rows, j, idxT, rT):
    for bl in range(128):
        vec = rows[j * 128 + bl, :]
        ivec = idxT[pl.ds(bl * LANES, LANES)]
        plsc.store_scatter(rT, [ivec], vec)


def _start_writes(out_hbm, rT, base, g, j, sem):
    f, bt = _unit_f_bt(base, g, j)
    pltpu.make_async_copy(rT.at[pl.ds(0, 1024)], out_hbm.at[f, 0, bt], sem).start()
    pltpu.make_async_copy(rT.at[pl.ds(1024, 1024)], out_hbm.at[f, 1, bt], sem).start()


def _wait_writes(out_hbm, rT, base, g, j, sem):
    f, bt = _unit_f_bt(base, g, j)
    pltpu.make_async_copy(rT.at[pl.ds(0, 1024)], out_hbm.at[f, 0, bt], sem).wait()
    pltpu.make_async_copy(rT.at[pl.ds(1024, 1024)], out_hbm.at[f, 1, bt], sem).wait()


def _body(xt_hbm, table_hbm, out_hbm, idxall, idxT, idxg0, idxg1, rows0, rows1,
          rT0a, rT0b, rT0c, rT0d, rT1a, rT1b, rT1c, rT1d,
          sg0, sg1, sw0a, sw0b, sw0c, sw0d, sw1a, sw1b, sw1c, sw1d):
    wid = lax.axis_index("s") * NC + lax.axis_index("c")
    base = wid * UPW
    pltpu.sync_copy(xt_hbm.at[pl.ds(base * 128, UPW * 128)], idxall)
    dv = lax.iota(jnp.int32, LANES) * 128

    def mk_idx(j, c):
        idxT[pl.ds(j * LANES, LANES)] = dv + j
        return c

    lax.fori_loop(0, 128, mk_idx, 0)

    rT0 = (rT0a, rT0b, rT0c, rT0d)
    rT1 = (rT1a, rT1b, rT1c, rT1d)
    sw0 = (sw0a, sw0b, sw0c, sw0d)
    sw1 = (sw1a, sw1b, sw1c, sw1d)

    _prep_idx(idxall, idxg0, base, 0)
    pltpu.make_async_copy(table_hbm.at[idxg0], rows0, sg0).start()

    def pair(k, carry):
        ga = 2 * k

        _prep_idx(idxall, idxg1, base, ga + 1)
        pltpu.make_async_copy(table_hbm.at[idxg1], rows1, sg1).start()

        pltpu.make_async_copy(table_hbm.at[idxg0], rows0, sg0).wait()
        for j in range(G):
            @pl.when(k > 0)
            def _():
                _wait_writes(out_hbm, rT0[j], base, ga - 2, j, sw0[j])

            _transpose_unit(rows0, j, idxT, rT0[j])
            _start_writes(out_hbm, rT0[j], base, ga, j, sw0[j])

        @pl.when(k < GPW // 2 - 1)
        def _():
            _prep_idx(idxall, idxg0, base, ga + 2)
            pltpu.make_async_copy(table_hbm.at[idxg0], rows0, sg0).start()

        pltpu.make_async_copy(table_hbm.at[idxg1], rows1, sg1).wait()
        for j in range(G):
            @pl.when(k > 0)
            def _():
                _wait_writes(out_hbm, rT1[j], base, ga - 1, j, sw1[j])

            _transpose_unit(rows1, j, idxT, rT1[j])
            _start_writes(out_hbm, rT1[j], base, ga + 1, j, sw1[j])

        return carry

    lax.fori_loop(0, GPW // 2, pair, 0)

    last = GPW - 2
    for j in range(G):
        _wait_writes(out_hbm, rT0[j], base, last, j, sw0[j])
        _wait_writes(out_hbm, rT1[j], base, last + 1, j, sw1[j])


@jax.jit
def _lookup(xt, table_lin):
    mesh = plsc.VectorSubcoreMesh(
        core_axis_name="c", subcore_axis_name="s", num_cores=NC, num_subcores=NS
    )
    return pl.kernel(
        _body,
        out_type=jax.ShapeDtypeStruct((F, 2, BT, 1024), jnp.float32),
        mesh=mesh,
        compiler_params=pltpu.CompilerParams(
            use_tc_tiling_on_sc=False, needs_layout_passes=False
        ),
        scratch_types=[
            pltpu.VMEM((UPW * 128,), jnp.int32),
            pltpu.VMEM((128 * LANES,), jnp.int32),
            pltpu.VMEM((G * 128,), jnp.int32),
            pltpu.VMEM((G * 128,), jnp.int32),
            pltpu.VMEM((G * 128, D), jnp.float32),
            pltpu.VMEM((G * 128, D), jnp.float32),
            pltpu.VMEM((D * 128,), jnp.float32),
            pltpu.VMEM((D * 128,), jnp.float32),
            pltpu.VMEM((D * 128,), jnp.float32),
            pltpu.VMEM((D * 128,), jnp.float32),
            pltpu.VMEM((D * 128,), jnp.float32),
            pltpu.VMEM((D * 128,), jnp.float32),
            pltpu.VMEM((D * 128,), jnp.float32),
            pltpu.VMEM((D * 128,), jnp.float32),
            pltpu.SemaphoreType.DMA,
            pltpu.SemaphoreType.DMA,
            pltpu.SemaphoreType.DMA,
            pltpu.SemaphoreType.DMA,
            pltpu.SemaphoreType.DMA,
            pltpu.SemaphoreType.DMA,
            pltpu.SemaphoreType.DMA,
            pltpu.SemaphoreType.DMA,
            pltpu.SemaphoreType.DMA,
            pltpu.SemaphoreType.DMA,
        ],
    )(xt, table_lin)


def kernel(x, table):
    tabT = table.T
    table_lin = _transpose_table(tabT, tabT.reshape(D * V))
    out5 = _lookup(x.T.reshape(F * B), table_lin).reshape(F, 2, BT, 8, 128)
    return out5.transpose(2, 4, 0, 1, 3).reshape(B, F, D)


# speedup vs baseline: 4.8882x; 4.8882x over previous
import jax
import jax.numpy as jnp
from jax import lax
from jax.experimental import pallas as pl
from jax.experimental.pallas import tpu as pltpu
from jax.experimental.pallas import tpu_sc as plsc

B = 16384
F = 26
D = 16
FIELD = 38462
V = F * FIELD
VPAD = 1000064

NC = 2
NS = 16
NW = NC * NS
BT = B // 128
UNITS = F * BT
UPW = UNITS // NW
G = 2
GPW = UPW // G
LANES = 16

CH = VPAD // 128
CPT = (CH + NW - 1) // NW


def _tr_chunk(slabs, lin, dtv, dsv):
    for v in range(128):
        vvec = jnp.full((LANES,), v, jnp.int32)
        vec = plsc.load_gather(slabs, [dtv, dsv, vvec])
        a = (v * 16) // 1024
        b = ((v * 16) % 1024) // 128
        lin[a, b, pl.ds((v % 8) * 16, LANES)] = vec


def _tr_load(tabTp, v0, slabs, sa, sb):
    pltpu.make_async_copy(tabTp.at[pl.ds(0, 8), pl.ds(v0, 128)],
                          slabs.at[0], sa).start()
    pltpu.make_async_copy(tabTp.at[pl.ds(8, 8), pl.ds(v0, 128)],
                          slabs.at[1], sb).start()


def _tr_load_wait(tabTp, v0, slabs, sa, sb):
    pltpu.make_async_copy(tabTp.at[pl.ds(0, 8), pl.ds(v0, 128)],
                          slabs.at[0], sa).wait()
    pltpu.make_async_copy(tabTp.at[pl.ds(8, 8), pl.ds(v0, 128)],
                          slabs.at[1], sb).wait()


def _tr_body(tabTp, out_hbm, slabs0, slabs1, lin0, lin1,
             s0a, s0b, s1a, s1b, sw0, sw1):
    wid = lax.axis_index("s") * NC + lax.axis_index("c")
    dvec = lax.iota(jnp.int32, LANES)
    dtv = lax.shift_right_logical(dvec, 3)
    dsv = lax.bitwise_and(dvec, 7)

    def v0_of(k):
        c = lax.min(k * NW + wid, CH - 1)
        return c * 128

    def r_of(k):
        c = lax.min(k * NW + wid, CH - 1)
        return c * 2

    _tr_load(tabTp, v0_of(0), slabs0, s0a, s0b)

    def pair(k, carry):
        a = 2 * k
        _tr_load(tabTp, v0_of(a + 1), slabs1, s1a, s1b)

        _tr_load_wait(tabTp, v0_of(a), slabs0, s0a, s0b)

        @pl.when(k > 0)
        def _():
            pltpu.make_async_copy(lin0, out_hbm.at[pl.ds(0, 2)], sw0).wait()

        _tr_chunk(slabs0, lin0, dtv, dsv)
        pltpu.make_async_copy(lin0, out_hbm.at[pl.ds(r_of(a), 2)], sw0).start()

        _tr_load(tabTp, v0_of(a + 2), slabs0, s0a, s0b)

        _tr_load_wait(tabTp, v0_of(a + 1), slabs1, s1a, s1b)

        @pl.when(k > 0)
        def _():
            pltpu.make_async_copy(lin1, out_hbm.at[pl.ds(0, 2)], sw1).wait()

        _tr_chunk(slabs1, lin1, dtv, dsv)
        pltpu.make_async_copy(lin1, out_hbm.at[pl.ds(r_of(a + 1), 2)], sw1).start()
        return carry

    lax.fori_loop(0, CPT // 2, pair, 0)

    last = CPT - 1
    _tr_load_wait(tabTp, v0_of(last), slabs0, s0a, s0b)
    pltpu.make_async_copy(lin0, out_hbm.at[pl.ds(0, 2)], sw0).wait()
    _tr_chunk(slabs0, lin0, dtv, dsv)
    pltpu.sync_copy(lin0, out_hbm.at[pl.ds(r_of(last), 2)])
    pltpu.make_async_copy(lin1, out_hbm.at[pl.ds(0, 2)], sw1).wait()


@jax.jit
def _transpose_table(tabTp):
    mesh = plsc.VectorSubcoreMesh(
        core_axis_name="c", subcore_axis_name="s", num_cores=NC, num_subcores=NS
    )
    return pl.kernel(
        _tr_body,
        out_type=jax.ShapeDtypeStruct((VPAD // 64, 8, 128), jnp.float32),
        mesh=mesh,
        compiler_params=pltpu.CompilerParams(
            use_tc_tiling_on_sc=True, needs_layout_passes=False
        ),
        scratch_types=[
            pltpu.VMEM((2, 8, 128), jnp.float32),
            pltpu.VMEM((2, 8, 128), jnp.float32),
            pltpu.VMEM((2, 8, 128), jnp.float32),
            pltpu.VMEM((2, 8, 128), jnp.float32),
            pltpu.SemaphoreType.DMA,
            pltpu.SemaphoreType.DMA,
            pltpu.SemaphoreType.DMA,
            pltpu.SemaphoreType.DMA,
            pltpu.SemaphoreType.DMA,
            pltpu.SemaphoreType.DMA,
        ],
    )(tabTp)


def _unit_f_bt(base, g, j):
    u = base + g * G + j
    return u // BT, u % BT


def _prep_idx(idxall, idxg, base, g):
    for j in range(G):
        f, _ = _unit_f_bt(base, g, j)
        off = f * FIELD
        for j2 in range(128 // LANES):
            s = pl.ds(g * (G * 128) + j * 128 + j2 * LANES, LANES)
            d = pl.ds(j * 128 + j2 * LANES, LANES)
            idxg[d] = idxall[s] + off


def _transpose_unit(rows, j, idxT, rT):
    for bl in range(128):
        vec = rows[j * 128 + bl, :]
        ivec = idxT[pl.ds(bl * LANES, LANES)]
        plsc.store_scatter(rT, [ivec], vec)


def _start_writes(out_hbm, rT, base, g, j, sem):
    f, bt = _unit_f_bt(base, g, j)
    pltpu.make_async_copy(rT.at[pl.ds(0, 1024)], out_hbm.at[f, 0, bt], sem).start()
    pltpu.make_async_copy(rT.at[pl.ds(1024, 1024)], out_hbm.at[f, 1, bt], sem).start()


def _wait_writes(out_hbm, rT, base, g, j, sem):
    f, bt = _unit_f_bt(base, g, j)
    pltpu.make_async_copy(rT.at[pl.ds(0, 1024)], out_hbm.at[f, 0, bt], sem).wait()
    pltpu.make_async_copy(rT.at[pl.ds(1024, 1024)], out_hbm.at[f, 1, bt], sem).wait()


def _body(xt_hbm, table_hbm, out_hbm, idxall, idxT, idxg0, idxg1, rows0, rows1,
          rT00, rT01, rT10, rT11, sg0, sg1, sw00, sw01, sw10, sw11):
    wid = lax.axis_index("s") * NC + lax.axis_index("c")
    base = wid * UPW
    pltpu.sync_copy(xt_hbm.at[pl.ds(base * 128, UPW * 128)], idxall)
    dv = lax.iota(jnp.int32, LANES) * 128

    def mk_idx(j, c):
        idxT[pl.ds(j * LANES, LANES)] = dv + j
        return c

    lax.fori_loop(0, 128, mk_idx, 0)

    rT0 = (rT00, rT01)
    rT1 = (rT10, rT11)
    sw0 = (sw00, sw01)
    sw1 = (sw10, sw11)

    _prep_idx(idxall, idxg0, base, 0)
    pltpu.make_async_copy(table_hbm.at[idxg0], rows0, sg0).start()

    def pair(k, carry):
        ga = 2 * k

        _prep_idx(idxall, idxg1, base, ga + 1)
        pltpu.make_async_copy(table_hbm.at[idxg1], rows1, sg1).start()

        pltpu.make_async_copy(table_hbm.at[idxg0], rows0, sg0).wait()
        for j in range(G):
            @pl.when(k > 0)
            def _():
                _wait_writes(out_hbm, rT0[j], base, ga - 2, j, sw0[j])

            _transpose_unit(rows0, j, idxT, rT0[j])
            _start_writes(out_hbm, rT0[j], base, ga, j, sw0[j])

        @pl.when(k < GPW // 2 - 1)
        def _():
            _prep_idx(idxall, idxg0, base, ga + 2)
            pltpu.make_async_copy(table_hbm.at[idxg0], rows0, sg0).start()

        pltpu.make_async_copy(table_hbm.at[idxg1], rows1, sg1).wait()
        for j in range(G):
            @pl.when(k > 0)
            def _():
                _wait_writes(out_hbm, rT1[j], base, ga - 1, j, sw1[j])

            _transpose_unit(rows1, j, idxT, rT1[j])
            _start_writes(out_hbm, rT1[j], base, ga + 1, j, sw1[j])

        return carry

    lax.fori_loop(0, GPW // 2, pair, 0)

    last = GPW - 2
    for j in range(G):
        _wait_writes(out_hbm, rT0[j], base, last, j, sw0[j])
        _wait_writes(out_hbm, rT1[j], base, last + 1, j, sw1[j])


@jax.jit
def _lookup(xt, table):
    mesh = plsc.VectorSubcoreMesh(
        core_axis_name="c", subcore_axis_name="s", num_cores=NC, num_subcores=NS
    )
    return pl.kernel(
        _body,
        out_type=jax.ShapeDtypeStruct((F, 2, BT, 1024), jnp.float32),
        mesh=mesh,
        compiler_params=pltpu.CompilerParams(
            use_tc_tiling_on_sc=False, needs_layout_passes=False
        ),
        scratch_types=[
            pltpu.VMEM((UPW * 128,), jnp.int32),
            pltpu.VMEM((128 * LANES,), jnp.int32),
            pltpu.VMEM((G * 128,), jnp.int32),
            pltpu.VMEM((G * 128,), jnp.int32),
            pltpu.VMEM((G * 128, D), jnp.float32),
            pltpu.VMEM((G * 128, D), jnp.float32),
            pltpu.VMEM((D * 128,), jnp.float32),
            pltpu.VMEM((D * 128,), jnp.float32),
            pltpu.VMEM((D * 128,), jnp.float32),
            pltpu.VMEM((D * 128,), jnp.float32),
            pltpu.SemaphoreType.DMA,
            pltpu.SemaphoreType.DMA,
            pltpu.SemaphoreType.DMA,
            pltpu.SemaphoreType.DMA,
            pltpu.SemaphoreType.DMA,
            pltpu.SemaphoreType.DMA,
        ],
    )(xt, table)


def kernel(x, table):
    tabTp = jnp.pad(table, ((0, VPAD - V), (0, 0))).T
    table_lin = _transpose_table(tabTp).reshape(VPAD, D)
    out5 = _lookup(x.T.reshape(F * B), table_lin).reshape(F, 2, BT, 8, 128)
    return out5.transpose(2, 4, 0, 1, 3).reshape(B, F, D)


# speedup vs baseline: 5.1623x; 1.0561x over previous
import jax
import jax.numpy as jnp
from jax import lax
from jax.experimental import pallas as pl
from jax.experimental.pallas import tpu as pltpu
from jax.experimental.pallas import tpu_sc as plsc

B = 16384
F = 26
D = 16
FIELD = 38462

NC = 2
NS = 16
NW = NC * NS
BT = B // 128
UNITS = F * BT
UPW = UNITS // NW
G = 2
GPW = UPW // G
LANES = 16


def _unit_f_bt(base, g, j):
    u = base + g * G + j
    return u // BT, u % BT


def _prep_idx(idxall, idxg, base, g):
    for j in range(G):
        f, _ = _unit_f_bt(base, g, j)
        off = f * FIELD
        for j2 in range(128 // LANES):
            s = pl.ds(g * (G * 128) + j * 128 + j2 * LANES, LANES)
            d = pl.ds(j * 128 + j2 * LANES, LANES)
            idxg[d] = idxall[s] + off


def _transpose_unit(rows, j, idxT, rT):
    for bl in range(128):
        vec = rows[j * 128 + bl, :]
        ivec = idxT[pl.ds(bl * LANES, LANES)]
        plsc.store_scatter(rT, [ivec], vec)


def _start_writes(out_hbm, rT, base, g, j, sem):
    f, bt = _unit_f_bt(base, g, j)
    pltpu.make_async_copy(rT.at[pl.ds(0, 1024)], out_hbm.at[f, 0, bt], sem).start()
    pltpu.make_async_copy(rT.at[pl.ds(1024, 1024)], out_hbm.at[f, 1, bt], sem).start()


def _wait_writes(out_hbm, rT, base, g, j, sem):
    f, bt = _unit_f_bt(base, g, j)
    pltpu.make_async_copy(rT.at[pl.ds(0, 1024)], out_hbm.at[f, 0, bt], sem).wait()
    pltpu.make_async_copy(rT.at[pl.ds(1024, 1024)], out_hbm.at[f, 1, bt], sem).wait()


def _body(xt_hbm, table_hbm, out_hbm, idxall, idxT, idxg0, idxg1, rows0, rows1,
          rT00, rT01, rT10, rT11, sg0, sg1, sw00, sw01, sw10, sw11):
    wid = lax.axis_index("s") * NC + lax.axis_index("c")
    base = wid * UPW
    pltpu.sync_copy(xt_hbm.at[pl.ds(base * 128, UPW * 128)], idxall)
    dv = lax.iota(jnp.int32, LANES) * 128

    def mk_idx(j, c):
        idxT[pl.ds(j * LANES, LANES)] = dv + j
        return c

    lax.fori_loop(0, 128, mk_idx, 0)

    rT0 = (rT00, rT01)
    rT1 = (rT10, rT11)
    sw0 = (sw00, sw01)
    sw1 = (sw10, sw11)

    _prep_idx(idxall, idxg0, base, 0)
    pltpu.make_async_copy(table_hbm.at[idxg0], rows0, sg0).start()

    def pair(k, carry):
        ga = 2 * k

        _prep_idx(idxall, idxg1, base, ga + 1)
        pltpu.make_async_copy(table_hbm.at[idxg1], rows1, sg1).start()

        pltpu.make_async_copy(table_hbm.at[idxg0], rows0, sg0).wait()
        for j in range(G):
            @pl.when(k > 0)
            def _():
                _wait_writes(out_hbm, rT0[j], base, ga - 2, j, sw0[j])

            _transpose_unit(rows0, j, idxT, rT0[j])
            _start_writes(out_hbm, rT0[j], base, ga, j, sw0[j])

        @pl.when(k < GPW // 2 - 1)
        def _():
            _prep_idx(idxall, idxg0, base, ga + 2)
            pltpu.make_async_copy(table_hbm.at[idxg0], rows0, sg0).start()

        pltpu.make_async_copy(table_hbm.at[idxg1], rows1, sg1).wait()
        for j in range(G):
            @pl.when(k > 0)
            def _():
                _wait_writes(out_hbm, rT1[j], base, ga - 1, j, sw1[j])

            _transpose_unit(rows1, j, idxT, rT1[j])
            _start_writes(out_hbm, rT1[j], base, ga + 1, j, sw1[j])

        return carry

    lax.fori_loop(0, GPW // 2, pair, 0)

    last = GPW - 2
    for j in range(G):
        _wait_writes(out_hbm, rT0[j], base, last, j, sw0[j])
        _wait_writes(out_hbm, rT1[j], base, last + 1, j, sw1[j])


@jax.jit
def _lookup(xt, table):
    mesh = plsc.VectorSubcoreMesh(
        core_axis_name="c", subcore_axis_name="s", num_cores=NC, num_subcores=NS
    )
    return pl.kernel(
        _body,
        out_type=jax.ShapeDtypeStruct((F, 2, BT, 1024), jnp.float32),
        mesh=mesh,
        compiler_params=pltpu.CompilerParams(
            use_tc_tiling_on_sc=False, needs_layout_passes=False
        ),
        scratch_types=[
            pltpu.VMEM((UPW * 128,), jnp.int32),
            pltpu.VMEM((128 * LANES,), jnp.int32),
            pltpu.VMEM((G * 128,), jnp.int32),
            pltpu.VMEM((G * 128,), jnp.int32),
            pltpu.VMEM((G * 128, D), jnp.float32),
            pltpu.VMEM((G * 128, D), jnp.float32),
            pltpu.VMEM((D * 128,), jnp.float32),
            pltpu.VMEM((D * 128,), jnp.float32),
            pltpu.VMEM((D * 128,), jnp.float32),
            pltpu.VMEM((D * 128,), jnp.float32),
            pltpu.SemaphoreType.DMA,
            pltpu.SemaphoreType.DMA,
            pltpu.SemaphoreType.DMA,
            pltpu.SemaphoreType.DMA,
            pltpu.SemaphoreType.DMA,
            pltpu.SemaphoreType.DMA,
        ],
    )(xt, table)


def kernel(x, table):
    out5 = _lookup(x.T.reshape(F * B), table).reshape(F, 2, BT, 8, 128)
    return out5.transpose(2, 4, 0, 1, 3).reshape(B, F, D)
